# baseline (speedup 1.0000x reference)
import jax
import jax.numpy as jnp
from jax import lax
from jax.experimental import pallas as pl
from jax.experimental.pallas import tpu as pltpu
from jax.experimental.pallas import tpu_sc as plsc

B = 4
K = 1024
DQ = 4
C = 32
HW = 1024
NT = 16
CHUNK = HW // (NT // B)

_DOT = dict(precision=lax.Precision.HIGHEST, preferred_element_type=jnp.float32)


def _scores_kernel(f_ref, q_ref, wf_ref, score_ref):
    Wf = wf_ref[:]
    Fall = f_ref[:].reshape(B * C, HW)
    Qall = q_ref[:].reshape(B * DQ, HW)
    Gall = lax.dot_general(Fall, Fall, (((1,), (1,)), ((), ())), **_DOT)
    Mall = lax.dot_general(Qall, Fall, (((1,), (1,)), ((), ())), **_DOT)

    sub = lax.broadcasted_iota(jnp.int32, (DQ * C, B * DQ * C), 0)
    lane = lax.broadcasted_iota(jnp.int32, (DQ * C, B * DQ * C), 1)
    keep = (sub // C) == ((lane // C) % DQ)
    gcols = []
    for b in range(B):
        Gb = Gall[b * C:(b + 1) * C, b * C:(b + 1) * C]
        grow = jnp.concatenate([Gb] * DQ, axis=1)
        gcols.append(jnp.concatenate([grow] * DQ, axis=0))
    Gbig = jnp.where(keep, jnp.concatenate(gcols, axis=1), 0.0)
    Y = lax.dot_general(Wf, Gbig, (((1,), (0,)), ((), ())), **_DOT)

    mparts = []
    for b in range(B):
        for q in range(DQ):
            mparts.append(Mall[b * DQ + q:b * DQ + q + 1, b * C:(b + 1) * C])
    Mbig = jnp.concatenate(mparts, axis=1)

    Wtile = jnp.concatenate([Wf] * B, axis=1)
    E = Wtile * (2.0 * Mbig - Y)

    s2 = lax.broadcasted_iota(jnp.int32, (B * DQ * C, B), 0)
    l2 = lax.broadcasted_iota(jnp.int32, (B * DQ * C, B), 1)
    selM = jnp.where((s2 // (DQ * C)) == l2, 1.0, 0.0)
    score_T = lax.dot_general(selM, E, (((0,), (1,)), ((), ())), **_DOT)
    score_ref[:] = score_T.reshape(B, 8, 128)


def _sc_select(score_hbm, f_hbm, q_hbm, wf_hbm,
               sel_hbm, codes_hbm, lp_hbm,
               score_v, idx_v, rows_v, out_v, fv, qv, selv, lpv, sem):
    wid = lax.axis_index("s")
    b = wid // (NT // B)
    ch = wid % (NT // B)

    pltpu.sync_copy(score_hbm.at[b], score_v)
    lanes = lax.iota(jnp.int32, 16)
    best = jnp.full((16,), -jnp.inf, jnp.float32)
    bidx = jnp.zeros((16,), jnp.int32)
    for r in range(8):
        for l in range(8):
            v = score_v[r, pl.ds(l * 16, 16)]
            kidx = lanes + (r * 128 + l * 16)
            upd = v > best
            best = jnp.where(upd, v, best)
            bidx = jnp.where(upd, kidx, bidx)
    _, sidx = plsc.sort_key_val(best, bidx, descending=True)
    idx_v[...] = sidx
    pltpu.async_copy(wf_hbm.at[idx_v], rows_v, sem).wait()

    @pl.when(ch == 0)
    def _():
        out_v[...] = sidx
        pltpu.sync_copy(out_v, codes_hbm.at[b, pl.ds(0, 16)])

    nrow = 32 // (NT // B) * 1
    pltpu.sync_copy(f_hbm.at[b, :, pl.ds(nrow * ch, nrow), :], fv)
    pltpu.sync_copy(q_hbm.at[b, :, pl.ds(nrow * ch, nrow), :], qv)
    lp = jnp.zeros((16,), jnp.float32)
    for q in range(DQ):
        wv0 = rows_v[0, pl.ds(q * C, 16)]
        wv1 = rows_v[0, pl.ds(q * C + 16, 16)]
        ws = [wv0[c] for c in range(16)] + [wv1[c] for c in range(16)]

        def jbody(j, lpc, _q=q, _ws=ws):
            jr = j // 2
            jc = (j % 2) * 16
            acc = _ws[0] * fv[0, jr, pl.ds(jc, 16)]
            for c in range(1, C):
                acc = acc + _ws[c] * fv[c, jr, pl.ds(jc, 16)]
            d = acc - qv[_q, jr, pl.ds(jc, 16)]
            selv[_q, jr, pl.ds(jc, 16)] = acc
            return lpc + d * d

        lp = lax.fori_loop(0, CHUNK // 16, jbody, lp)
    lpv[...] = lp
    pltpu.sync_copy(selv, sel_hbm.at[b, :, pl.ds(nrow * ch, nrow), :])
    pltpu.sync_copy(lpv, lp_hbm.at[wid])


def kernel(features, query, W):
    wf = W.reshape(K, DQ * C)

    score3 = pl.pallas_call(
        _scores_kernel,
        out_shape=jax.ShapeDtypeStruct((B, 8, 128), jnp.float32),
        out_specs=pl.BlockSpec(memory_space=pltpu.VMEM),
        in_specs=[pl.BlockSpec(memory_space=pltpu.VMEM)] * 3,
    )(features, query, wf)

    mesh = plsc.VectorSubcoreMesh(
        core_axis_name="c", subcore_axis_name="s", num_cores=1
    )
    nrow = 32 // (NT // B)
    sel, codes8, lparts = pl.kernel(
        _sc_select,
        out_type=[
            jax.ShapeDtypeStruct((B, DQ, 32, 32), jnp.float32),
            jax.ShapeDtypeStruct((8, 128), jnp.int32),
            jax.ShapeDtypeStruct((NT, 16), jnp.float32),
        ],
        mesh=mesh,
        compiler_params=pltpu.CompilerParams(needs_layout_passes=False),
        scratch_types=[
            pltpu.VMEM((8, 128), jnp.float32),
            pltpu.VMEM((16,), jnp.int32),
            pltpu.VMEM((16, 128), jnp.float32),
            pltpu.VMEM((16,), jnp.int32),
            pltpu.VMEM((C, nrow, 32), jnp.float32),
            pltpu.VMEM((DQ, nrow, 32), jnp.float32),
            pltpu.VMEM((DQ, nrow, 32), jnp.float32),
            pltpu.VMEM((16,), jnp.float32),
            pltpu.SemaphoreType.DMA,
        ],
    )(score3, features, query, wf)

    codes = codes8[:B, 0]
    loss = jnp.sum(lparts) / jnp.float32(B * DQ * HW)
    return sel, codes, loss


# speedup vs baseline: 1.2190x; 1.2190x over previous
import functools

import jax
import jax.numpy as jnp
from jax import lax
from jax.experimental import pallas as pl
from jax.experimental.pallas import tpu as pltpu
from jax.experimental.pallas import tpu_sc as plsc

B = 4
K = 1024
DQ = 4
C = 32
HW = 1024

_DOT = dict(precision=lax.Precision.HIGHEST, preferred_element_type=jnp.float32)


def _scores_kernel(f_ref, q_ref, wf_ref, score_ref):
    Wf = wf_ref[:]
    Fall = f_ref[:].reshape(B * C, HW)
    Qall = q_ref[:].reshape(B * DQ, HW)
    Gall = lax.dot_general(Fall, Fall, (((1,), (1,)), ((), ())), **_DOT)
    Mall = lax.dot_general(Qall, Fall, (((1,), (1,)), ((), ())), **_DOT)

    sub = lax.broadcasted_iota(jnp.int32, (DQ * C, B * DQ * C), 0)
    lane = lax.broadcasted_iota(jnp.int32, (DQ * C, B * DQ * C), 1)
    keep = (sub // C) == ((lane // C) % DQ)
    gcols = []
    for b in range(B):
        Gb = Gall[b * C:(b + 1) * C, b * C:(b + 1) * C]
        grow = jnp.concatenate([Gb] * DQ, axis=1)
        gcols.append(jnp.concatenate([grow] * DQ, axis=0))
    Gbig = jnp.where(keep, jnp.concatenate(gcols, axis=1), 0.0)
    Y = lax.dot_general(Wf, Gbig, (((1,), (0,)), ((), ())), **_DOT)

    mparts = []
    for b in range(B):
        for q in range(DQ):
            mparts.append(Mall[b * DQ + q:b * DQ + q + 1, b * C:(b + 1) * C])
    Mbig = jnp.concatenate(mparts, axis=1)

    Wtile = jnp.concatenate([Wf] * B, axis=1)
    E = Wtile * (2.0 * Mbig - Y)

    s2 = lax.broadcasted_iota(jnp.int32, (B * DQ * C, B), 0)
    l2 = lax.broadcasted_iota(jnp.int32, (B * DQ * C, B), 1)
    selM = jnp.where((s2 // (DQ * C)) == l2, 1.0, 0.0)
    score_T = lax.dot_general(selM, E, (((0,), (1,)), ((), ())), **_DOT)
    score_ref[:] = score_T.reshape(B, 8, 128)


def _sc_select(score_hbm, wf_hbm, wsel_hbm, codes_hbm,
               score_v, idx_v, rows_v, out_v, sem):
    wid = lax.axis_index("s")

    @pl.when(wid < B)
    def _():
        pltpu.sync_copy(score_hbm.at[wid], score_v)
        lanes = lax.iota(jnp.int32, 16)
        best = jnp.full((16,), -jnp.inf, jnp.float32)
        bidx = jnp.zeros((16,), jnp.int32)
        for r in range(8):
            for l in range(8):
                v = score_v[r, pl.ds(l * 16, 16)]
                kidx = lanes + (r * 128 + l * 16)
                upd = v > best
                best = jnp.where(upd, v, best)
                bidx = jnp.where(upd, kidx, bidx)
        _, sidx = plsc.sort_key_val(best, bidx, descending=True)
        idx_v[...] = sidx
        out_v[...] = sidx
        pltpu.async_copy(wf_hbm.at[idx_v], rows_v, sem).wait()
        pltpu.sync_copy(rows_v.at[0], wsel_hbm.at[wid])
        pltpu.sync_copy(out_v, codes_hbm.at[wid, pl.ds(0, 16)])


def _select_kernel(f_ref, q_ref, wsel_ref, codes_in_ref,
                   sel_ref, codes_ref, loss_ref):
    Fall = f_ref[:].reshape(B * C, HW)
    Qall = q_ref[:].reshape(B * DQ, HW)
    zer = jnp.zeros((DQ, DQ * C), jnp.float32)
    wrows = []
    for b in range(B):
        wrow = wsel_ref[pl.ds(b, 1), :]
        Wsel = jnp.concatenate(
            [wrow[:, q * C:(q + 1) * C] for q in range(DQ)], axis=0
        )
        pads = [zer[:, :b * C], Wsel, zer[:, (b + 1) * C:]]
        wrows.append(jnp.concatenate([p for p in pads if p.shape[1]], axis=1))
        codes_ref[b] = codes_in_ref[b, 0]
    Wbig = jnp.concatenate(wrows, axis=0)
    selall = lax.dot_general(Wbig, Fall, (((1,), (0,)), ((), ())), **_DOT)
    sel_ref[:] = selall.reshape(B, DQ, 32, 32)
    loss_ref[...] = jnp.sum((selall - Qall) ** 2) / jnp.float32(B * DQ * HW)


def kernel(features, query, W):
    wf = W.reshape(K, DQ * C)

    score3 = pl.pallas_call(
        _scores_kernel,
        out_shape=jax.ShapeDtypeStruct((B, 8, 128), jnp.float32),
        out_specs=pl.BlockSpec(memory_space=pltpu.VMEM),
        in_specs=[pl.BlockSpec(memory_space=pltpu.VMEM)] * 3,
    )(features, query, wf)

    mesh = plsc.VectorSubcoreMesh(
        core_axis_name="c", subcore_axis_name="s", num_cores=1
    )
    wsel8, codes8 = pl.kernel(
        _sc_select,
        out_type=[
            jax.ShapeDtypeStruct((8, 128), jnp.float32),
            jax.ShapeDtypeStruct((8, 128), jnp.int32),
        ],
        mesh=mesh,
        compiler_params=pltpu.CompilerParams(needs_layout_passes=False),
        scratch_types=[
            pltpu.VMEM((8, 128), jnp.float32),
            pltpu.VMEM((16,), jnp.int32),
            pltpu.VMEM((16, 128), jnp.float32),
            pltpu.VMEM((16,), jnp.int32),
            pltpu.SemaphoreType.DMA,
        ],
    )(score3, wf)

    sel, codes, loss = pl.pallas_call(
        _select_kernel,
        out_shape=[
            jax.ShapeDtypeStruct((B, DQ, 32, 32), jnp.float32),
            jax.ShapeDtypeStruct((B,), jnp.int32),
            jax.ShapeDtypeStruct((), jnp.float32),
        ],
        out_specs=[
            pl.BlockSpec(memory_space=pltpu.VMEM),
            pl.BlockSpec(memory_space=pltpu.SMEM),
            pl.BlockSpec(memory_space=pltpu.SMEM),
        ],
        in_specs=[pl.BlockSpec(memory_space=pltpu.VMEM)] * 4,
    )(features, query, wsel8, codes8)
    return sel, codes, loss
